# baseline (speedup 1.0000x reference)
import functools

import jax
import jax.numpy as jnp
from jax import lax
from jax.experimental import pallas as pl
from jax.experimental.pallas import tpu as pltpu
from jax.experimental.pallas import tpu_sc as plsc

E = 16
K = 2
D = 1024
F = 2048
T = 2048
M = T * K
TM = 128
NT = M // TM + E
MP = NT * TM
IDS_R = M // 128
IDS_C = 128

NC = 2
NS = 16
NW = NC * NS


def _routing_body(ids_ref, pos_ref, end_ref):
    ids = ids_ref[...]
    li = lax.broadcasted_iota(jnp.int32, (IDS_C, IDS_C), 0)
    lj = lax.broadcasted_iota(jnp.int32, (IDS_C, IDS_C), 1)
    lmat = (li <= lj).astype(jnp.float32)
    ri = lax.broadcasted_iota(jnp.int32, (IDS_R, IDS_R), 0)
    rj = lax.broadcasted_iota(jnp.int32, (IDS_R, IDS_R), 1)
    tstrict = (rj < ri).astype(jnp.float32)
    pos = jnp.zeros((IDS_R, IDS_C), jnp.int32)
    off = jnp.int32(0)
    ends = []
    for e in range(E):
        m = ids == e
        mf = m.astype(jnp.float32)
        s = jnp.dot(mf, lmat, preferred_element_type=jnp.float32)
        row_tot = s[:, IDS_C - 1:IDS_C]
        excl = jnp.dot(tstrict, row_tot, preferred_element_type=jnp.float32)
        rank = (s - mf + excl).astype(jnp.int32)
        pos = jnp.where(m, off + rank, pos)
        tot = jnp.sum(mf).astype(jnp.int32)
        off = off + ((tot + TM - 1) // TM) * TM
        ends.append(off)
    pos_ref[...] = pos
    lane = lax.broadcasted_iota(jnp.int32, (1, 128), 1)
    endv = jnp.zeros((1, 128), jnp.int32)
    for e in range(E):
        endv = jnp.where(lane == e, ends[e], endv)
    end_ref[...] = endv


_routing = pl.pallas_call(
    _routing_body,
    out_shape=(
        jax.ShapeDtypeStruct((IDS_R, IDS_C), jnp.int32),
        jax.ShapeDtypeStruct((1, 128), jnp.int32),
    ),
)


def _gmm_body(meta_ref, x_ref, w0_hbm, w1_hbm, wo_hbm, o_ref,
              w0b, w1b, wob, *sems):
    t = pl.program_id(0)
    rs = meta_ref[0, t]
    p = meta_ref[1, t]
    nre = meta_ref[2, t]
    hn = meta_ref[3, t]
    cur = meta_ref[4, t]

    def _copies(e_idx, pp):
        return (
            pltpu.make_async_copy(w0_hbm.at[e_idx], w0b.at[pp], sems[pp]),
            pltpu.make_async_copy(w1_hbm.at[e_idx], w1b.at[pp], sems[2 + pp]),
            pltpu.make_async_copy(wo_hbm.at[e_idx], wob.at[pp], sems[4 + pp]),
        )

    def _issue(e_idx, pp):
        for c in _copies(e_idx, pp):
            c.start()

    def _wait(pp):
        for c in _copies(0, pp):
            c.wait()

    @pl.when(t == 0)
    def _():
        _issue(cur, 0)

    @pl.when((rs == 1) & (hn == 1) & (p == 0))
    def _():
        _issue(nre, 1)

    @pl.when((rs == 1) & (hn == 1) & (p == 1))
    def _():
        _issue(nre, 0)

    @pl.when((rs == 1) & (p == 0))
    def _():
        _wait(0)

    @pl.when((rs == 1) & (p == 1))
    def _():
        _wait(1)

    @pl.when(meta_ref[5, t] == 1)
    def _():
        x = x_ref[...]
        a = jnp.dot(x, w0b[p], preferred_element_type=jnp.float32)
        b = jnp.dot(x, w1b[p], preferred_element_type=jnp.float32)
        h = a * jax.nn.sigmoid(a) * b
        o_ref[...] = jnp.dot(h, wob[p], preferred_element_type=jnp.float32)


_gmm = pl.pallas_call(
    _gmm_body,
    grid_spec=pltpu.PrefetchScalarGridSpec(
        num_scalar_prefetch=1,
        grid=(NT,),
        in_specs=[
            pl.BlockSpec((TM, D), lambda t, meta: (t, 0)),
            pl.BlockSpec(memory_space=pl.ANY),
            pl.BlockSpec(memory_space=pl.ANY),
            pl.BlockSpec(memory_space=pl.ANY),
        ],
        out_specs=pl.BlockSpec((TM, D), lambda t, meta: (t, 0)),
        scratch_shapes=[
            pltpu.VMEM((2, D, F), jnp.float32),
            pltpu.VMEM((2, D, F), jnp.float32),
            pltpu.VMEM((2, F, D), jnp.float32),
        ] + [pltpu.SemaphoreType.DMA] * 6,
    ),
    out_shape=jax.ShapeDtypeStruct((MP, D), jnp.float32),
)


def _gmm_meta(ends):
    tvals = jnp.arange(NT, dtype=jnp.int32) * TM
    eot = jnp.minimum(
        jnp.sum((tvals[:, None] >= ends[None, :]).astype(jnp.int32), axis=1),
        E - 1)
    rs = jnp.concatenate([jnp.ones((1,), jnp.int32),
                          (eot[1:] != eot[:-1]).astype(jnp.int32)])
    run_id = jnp.cumsum(rs) - 1
    parity = run_id % 2
    s_idx = jnp.arange(NT, dtype=jnp.int32)
    cand = jnp.where((rs[None, :] == 1) & (s_idx[None, :] > s_idx[:, None]),
                     s_idx[None, :], NT)
    ncp = jnp.min(cand, axis=1)
    hn = (ncp < NT).astype(jnp.int32)
    nre = eot[jnp.minimum(ncp, NT - 1)]
    valid = (tvals < ends[E - 1]).astype(jnp.int32)
    return jnp.stack([rs, parity, nre, hn, eot, valid]).astype(jnp.int32)


_ROWS_W = T // NW
_GROWS_W = M // NW
_GCHUNK = 64


@functools.cache
def _sc_kernels():
    mesh = plsc.VectorSubcoreMesh(
        core_axis_name="c", subcore_axis_name="s",
        num_cores=NC, num_subcores=NS)

    @functools.partial(
        pl.kernel,
        out_type=jax.ShapeDtypeStruct((MP, D), jnp.float32),
        mesh=mesh,
        scratch_types=[
            pltpu.VMEM((_ROWS_W, D), jnp.float32),
            pltpu.VMEM((_ROWS_W,), jnp.int32),
            pltpu.VMEM((_ROWS_W,), jnp.int32),
            pltpu.SemaphoreType.DMA,
            pltpu.SemaphoreType.DMA,
        ],
    )
    def sc_permute(hid_hbm, pe_hbm, po_hbm, xp_hbm, rows_v, i0_v, i1_v,
                   sem0, sem1):
        wid = lax.axis_index("s") * NC + lax.axis_index("c")
        base = wid * _ROWS_W
        pltpu.sync_copy(hid_hbm.at[pl.ds(base, _ROWS_W)], rows_v)
        pltpu.sync_copy(pe_hbm.at[pl.ds(base, _ROWS_W)], i0_v)
        pltpu.sync_copy(po_hbm.at[pl.ds(base, _ROWS_W)], i1_v)
        c0 = pltpu.async_copy(rows_v, xp_hbm.at[i0_v], sem0)
        c1 = pltpu.async_copy(rows_v, xp_hbm.at[i1_v], sem1)
        c0.wait()
        c1.wait()

    tc = 16

    @functools.partial(
        pl.kernel,
        out_type=jax.ShapeDtypeStruct((T, D), jnp.float32),
        mesh=mesh,
        scratch_types=[
            pltpu.VMEM((tc,), jnp.int32),
            pltpu.VMEM((tc,), jnp.int32),
            pltpu.VMEM((tc, D), jnp.float32),
            pltpu.VMEM((tc, D), jnp.float32),
            pltpu.VMEM((tc, D), jnp.float32),
            pltpu.VMEM((tc,), jnp.float32),
            pltpu.VMEM((tc,), jnp.float32),
            pltpu.SemaphoreType.DMA,
            pltpu.SemaphoreType.DMA,
        ],
    )
    def sc_combine(out2_hbm, pe_hbm, po_hbm, tw0_hbm, tw1_hbm, out_hbm,
                   ie_v, io_v, re_v, ro_v, ou_v, we_v, wo_v, sem_e, sem_o):
        wid = lax.axis_index("s") * NC + lax.axis_index("c")
        for c in range(_ROWS_W // tc):
            base = wid * _ROWS_W + c * tc
            pltpu.sync_copy(pe_hbm.at[pl.ds(base, tc)], ie_v)
            pltpu.sync_copy(po_hbm.at[pl.ds(base, tc)], io_v)
            pltpu.sync_copy(tw0_hbm.at[pl.ds(base, tc)], we_v)
            pltpu.sync_copy(tw1_hbm.at[pl.ds(base, tc)], wo_v)
            ce = pltpu.async_copy(out2_hbm.at[ie_v], re_v, sem_e)
            co = pltpu.async_copy(out2_hbm.at[io_v], ro_v, sem_o)
            ce.wait()
            co.wait()

            dn = lax.GatherDimensionNumbers(
                offset_dims=(), collapsed_slice_dims=(0,),
                start_index_map=(0,))

            def _bcast(vec, j):
                idx = (jnp.zeros((16,), jnp.int32) + j)[:, None]
                return lax.gather(
                    vec, idx, dn, slice_sizes=(1,),
                    mode=lax.GatherScatterMode.PROMISE_IN_BOUNDS)

            for half in range(tc // 16):
                wev = we_v[pl.ds(half * 16, 16)]
                wov = wo_v[pl.ds(half * 16, 16)]

                def jbody(j16, carry, _h=half, _we=wev, _wo=wov):
                    j = _h * 16 + j16
                    w0b = _bcast(_we, j16)
                    w1b = _bcast(_wo, j16)

                    def qbody(q, carry2):
                        for u in range(8):
                            sl = pl.ds((q * 8 + u) * 16, 16)
                            ou_v[j, sl] = (re_v[j, sl] * w0b
                                           + ro_v[j, sl] * w1b)
                        return carry2

                    lax.fori_loop(0, D // 128, qbody, 0)
                    return carry

                lax.fori_loop(0, 16, jbody, 0)
            pltpu.sync_copy(ou_v, out_hbm.at[pl.ds(base, tc)])

    return sc_permute, sc_combine


def kernel(hidden_states, topk_weights, topk_ids, wi_0, wi_1, wo):
    ids2 = topk_ids.reshape(IDS_R, IDS_C)
    pos, ends_v = _routing(ids2)
    pos_flat = pos.reshape(M)
    pos_even = pos_flat[0::2]
    pos_odd = pos_flat[1::2]
    ends = ends_v.reshape(128)[:E]
    sc_permute, sc_combine = _sc_kernels()
    x_pad = sc_permute(hidden_states, pos_even, pos_odd)
    meta = _gmm_meta(ends)
    out2 = _gmm(meta, x_pad, wi_0, wi_1, wo)
    tw0 = topk_weights[:, 0]
    tw1 = topk_weights[:, 1]
    return sc_combine(out2, pos_even, pos_odd, tw0, tw1)
    unsorted = sc_gather(out2, pos_flat)
    u2 = unsorted.reshape(T, K * D)
    return _combine(u2, topk_weights)


# speedup vs baseline: 1.0581x; 1.0581x over previous
import functools

import jax
import jax.numpy as jnp
from jax import lax
from jax.experimental import pallas as pl
from jax.experimental.pallas import tpu as pltpu
from jax.experimental.pallas import tpu_sc as plsc

E = 16
K = 2
D = 1024
F = 2048
T = 2048
M = T * K
TM = 128
NT = M // TM + E
MP = NT * TM
IDS_R = M // 128
IDS_C = 128

NC = 2
NS = 16
NW = NC * NS


def _routing_body(ids_ref, pos_ref, end_ref):
    ids = ids_ref[...]
    li = lax.broadcasted_iota(jnp.int32, (IDS_C, IDS_C), 0)
    lj = lax.broadcasted_iota(jnp.int32, (IDS_C, IDS_C), 1)
    lmat = (li <= lj).astype(jnp.float32)
    ri = lax.broadcasted_iota(jnp.int32, (IDS_R, IDS_R), 0)
    rj = lax.broadcasted_iota(jnp.int32, (IDS_R, IDS_R), 1)
    tstrict = (rj < ri).astype(jnp.float32)
    pos = jnp.zeros((IDS_R, IDS_C), jnp.int32)
    off = jnp.int32(0)
    ends = []
    for e in range(E):
        m = ids == e
        mf = m.astype(jnp.float32)
        s = jnp.dot(mf, lmat, preferred_element_type=jnp.float32)
        row_tot = s[:, IDS_C - 1:IDS_C]
        excl = jnp.dot(tstrict, row_tot, preferred_element_type=jnp.float32)
        rank = (s - mf + excl).astype(jnp.int32)
        pos = jnp.where(m, off + rank, pos)
        tot = jnp.sum(mf).astype(jnp.int32)
        off = off + ((tot + TM - 1) // TM) * TM
        ends.append(off)
    pos_ref[...] = pos
    lane = lax.broadcasted_iota(jnp.int32, (1, 128), 1)
    endv = jnp.zeros((1, 128), jnp.int32)
    for e in range(E):
        endv = jnp.where(lane == e, ends[e], endv)
    end_ref[...] = endv


_routing = pl.pallas_call(
    _routing_body,
    out_shape=(
        jax.ShapeDtypeStruct((IDS_R, IDS_C), jnp.int32),
        jax.ShapeDtypeStruct((1, 128), jnp.int32),
    ),
)


def _gmm_body(meta_ref, x_ref, w0_hbm, w1_hbm, wo_hbm, o_ref,
              w0b, w1b, wob, *sems):
    t = pl.program_id(0)
    rs = meta_ref[0, t]
    p = meta_ref[1, t]
    nre = meta_ref[2, t]
    hn = meta_ref[3, t]
    cur = meta_ref[4, t]

    def _copies(e_idx, pp):
        return (
            pltpu.make_async_copy(w0_hbm.at[e_idx], w0b.at[pp], sems[pp]),
            pltpu.make_async_copy(w1_hbm.at[e_idx], w1b.at[pp], sems[2 + pp]),
            pltpu.make_async_copy(wo_hbm.at[e_idx], wob.at[pp], sems[4 + pp]),
        )

    def _issue(e_idx, pp):
        for c in _copies(e_idx, pp):
            c.start()

    def _wait(pp):
        for c in _copies(0, pp):
            c.wait()

    @pl.when(t == 0)
    def _():
        _issue(cur, 0)

    @pl.when((rs == 1) & (hn == 1) & (p == 0))
    def _():
        _issue(nre, 1)

    @pl.when((rs == 1) & (hn == 1) & (p == 1))
    def _():
        _issue(nre, 0)

    @pl.when((rs == 1) & (p == 0))
    def _():
        _wait(0)

    @pl.when((rs == 1) & (p == 1))
    def _():
        _wait(1)

    @pl.when(meta_ref[5, t] == 1)
    def _():
        x = x_ref[...]
        a = jnp.dot(x, w0b[p], preferred_element_type=jnp.float32)
        b = jnp.dot(x, w1b[p], preferred_element_type=jnp.float32)
        h = a * jax.nn.sigmoid(a) * b
        o_ref[...] = jnp.dot(h, wob[p], preferred_element_type=jnp.float32)


_gmm = pl.pallas_call(
    _gmm_body,
    grid_spec=pltpu.PrefetchScalarGridSpec(
        num_scalar_prefetch=1,
        grid=(NT,),
        in_specs=[
            pl.BlockSpec((TM, D), lambda t, meta: (t, 0)),
            pl.BlockSpec(memory_space=pl.ANY),
            pl.BlockSpec(memory_space=pl.ANY),
            pl.BlockSpec(memory_space=pl.ANY),
        ],
        out_specs=pl.BlockSpec((TM, D), lambda t, meta: (t, 0)),
        scratch_shapes=[
            pltpu.VMEM((2, D, F), jnp.float32),
            pltpu.VMEM((2, D, F), jnp.float32),
            pltpu.VMEM((2, F, D), jnp.float32),
        ] + [pltpu.SemaphoreType.DMA] * 6,
    ),
    out_shape=jax.ShapeDtypeStruct((MP, D), jnp.float32),
)


def _gmm_meta(ends):
    tvals = jnp.arange(NT, dtype=jnp.int32) * TM
    eot = jnp.minimum(
        jnp.sum((tvals[:, None] >= ends[None, :]).astype(jnp.int32), axis=1),
        E - 1)
    rs = jnp.concatenate([jnp.ones((1,), jnp.int32),
                          (eot[1:] != eot[:-1]).astype(jnp.int32)])
    run_id = jnp.cumsum(rs) - 1
    parity = run_id % 2
    s_idx = jnp.arange(NT, dtype=jnp.int32)
    cand = jnp.where((rs[None, :] == 1) & (s_idx[None, :] > s_idx[:, None]),
                     s_idx[None, :], NT)
    ncp = jnp.min(cand, axis=1)
    hn = (ncp < NT).astype(jnp.int32)
    nre = eot[jnp.minimum(ncp, NT - 1)]
    valid = (tvals < ends[E - 1]).astype(jnp.int32)
    return jnp.stack([rs, parity, nre, hn, eot, valid]).astype(jnp.int32)


_ROWS_W = T // NW
_GROWS_W = M // NW
_GCHUNK = 64


@functools.cache
def _sc_kernels():
    mesh = plsc.VectorSubcoreMesh(
        core_axis_name="c", subcore_axis_name="s",
        num_cores=NC, num_subcores=NS)

    @functools.partial(
        pl.kernel,
        out_type=jax.ShapeDtypeStruct((MP, D), jnp.float32),
        mesh=mesh,
        scratch_types=[
            pltpu.VMEM((_ROWS_W, D), jnp.float32),
            pltpu.VMEM((_ROWS_W,), jnp.int32),
            pltpu.VMEM((_ROWS_W,), jnp.int32),
            pltpu.SemaphoreType.DMA,
            pltpu.SemaphoreType.DMA,
        ],
    )
    def sc_permute(hid_hbm, pe_hbm, po_hbm, xp_hbm, rows_v, i0_v, i1_v,
                   sem0, sem1):
        wid = lax.axis_index("s") * NC + lax.axis_index("c")
        base = wid * _ROWS_W
        pltpu.sync_copy(hid_hbm.at[pl.ds(base, _ROWS_W)], rows_v)
        pltpu.sync_copy(pe_hbm.at[pl.ds(base, _ROWS_W)], i0_v)
        pltpu.sync_copy(po_hbm.at[pl.ds(base, _ROWS_W)], i1_v)
        pltpu.async_copy(rows_v, xp_hbm.at[i0_v], sem0).wait()
        pltpu.async_copy(rows_v, xp_hbm.at[i1_v], sem1).wait()

    tc = 16

    @functools.partial(
        pl.kernel,
        out_type=jax.ShapeDtypeStruct((T, D), jnp.float32),
        mesh=mesh,
        scratch_types=[
            pltpu.VMEM((tc,), jnp.int32),
            pltpu.VMEM((tc,), jnp.int32),
            pltpu.VMEM((tc, D), jnp.float32),
            pltpu.VMEM((tc, D), jnp.float32),
            pltpu.VMEM((tc, D), jnp.float32),
            pltpu.VMEM((tc,), jnp.float32),
            pltpu.VMEM((tc,), jnp.float32),
            pltpu.SemaphoreType.DMA,
            pltpu.SemaphoreType.DMA,
        ],
    )
    def sc_combine(out2_hbm, pe_hbm, po_hbm, tw0_hbm, tw1_hbm, out_hbm,
                   ie_v, io_v, re_v, ro_v, ou_v, we_v, wo_v, sem_e, sem_o):
        wid = lax.axis_index("s") * NC + lax.axis_index("c")
        for c in range(_ROWS_W // tc):
            base = wid * _ROWS_W + c * tc
            pltpu.sync_copy(pe_hbm.at[pl.ds(base, tc)], ie_v)
            pltpu.sync_copy(po_hbm.at[pl.ds(base, tc)], io_v)
            pltpu.sync_copy(tw0_hbm.at[pl.ds(base, tc)], we_v)
            pltpu.sync_copy(tw1_hbm.at[pl.ds(base, tc)], wo_v)
            ce = pltpu.async_copy(out2_hbm.at[ie_v], re_v, sem_e)
            co = pltpu.async_copy(out2_hbm.at[io_v], ro_v, sem_o)
            ce.wait()
            co.wait()

            dn = lax.GatherDimensionNumbers(
                offset_dims=(), collapsed_slice_dims=(0,),
                start_index_map=(0,))

            def _bcast(vec, j):
                idx = (jnp.zeros((16,), jnp.int32) + j)[:, None]
                return lax.gather(
                    vec, idx, dn, slice_sizes=(1,),
                    mode=lax.GatherScatterMode.PROMISE_IN_BOUNDS)

            wev = we_v[...]
            wov = wo_v[...]

            def jbody(j, carry):
                w0b = _bcast(wev, j)
                w1b = _bcast(wov, j)

                def qbody(q, carry2):
                    sl = pl.ds(q * 16, 16)
                    ou_v[j, sl] = re_v[j, sl] * w0b + ro_v[j, sl] * w1b
                    return carry2

                lax.fori_loop(0, D // 16, qbody, 0)
                return carry

            lax.fori_loop(0, tc, jbody, 0)
            pltpu.sync_copy(ou_v, out_hbm.at[pl.ds(base, tc)])

    return sc_permute, sc_combine


def kernel(hidden_states, topk_weights, topk_ids, wi_0, wi_1, wo):
    ids2 = topk_ids.reshape(IDS_R, IDS_C)
    pos, ends_v = _routing(ids2)
    pos_flat = pos.reshape(M)
    pos_even = pos_flat[0::2]
    pos_odd = pos_flat[1::2]
    ends = ends_v.reshape(128)[:E]
    sc_permute, sc_combine = _sc_kernels()
    x_pad = sc_permute(hidden_states, pos_even, pos_odd)
    meta = _gmm_meta(ends)
    out2 = _gmm(meta, x_pad, wi_0, wi_1, wo)
    tw0 = topk_weights[:, 0]
    tw1 = topk_weights[:, 1]
    return sc_combine(out2, pos_even, pos_odd, tw0, tw1)
    unsorted = sc_gather(out2, pos_flat)
    u2 = unsorted.reshape(T, K * D)
    return _combine(u2, topk_weights)
